# baseline (speedup 1.0000x reference)
import jax
import jax.numpy as jnp
from jax import lax
from jax.experimental import pallas as pl
from jax.experimental.pallas import tpu as pltpu
from jax.experimental.pallas import tpu_sc as plsc

E = 50000
T = 800000
R = 512
D = 768
B = 32
L = 32
STEPS = 2

NC = 2
NS = 16
LANES = 16

CH = 8000
NCH = T // CH
UNROLL = 10


def _question_body(qe_ref, qwh_ref, mask_ref, w0_ref, b0_ref, w1_ref, b1_ref,
                   wr_ref, br_ref, wh_ref, bh_ref, sj_ref, rl_ref, ob_ref,
                   wa0_ref, wa1_ref, rel0_ref, rel1_ref, hop_ref, pack_ref,
                   ob2_ref, hb0_ref, hb1_ref):
    pack_ref[...] = sj_ref[...] | (rl_ref[...] << 16)
    ob2_ref[...] = ob_ref[:, 0, :] | (ob_ref[:, 1, :] << 16)
    qe = qe_ref[...]
    qwh = qwh_ref[...]
    msk = mask_ref[...]
    steps = ((w0_ref, b0_ref, wa0_ref, rel0_ref),
             (w1_ref, b1_ref, wa1_ref, rel1_ref))
    for w_ref, b_ref, wa_out, rel_out in steps:
        cq = jnp.tanh(
            jnp.dot(qe, w_ref[...], preferred_element_type=jnp.float32)
            + b_ref[...])
        logits = jnp.sum(cq[:, None, :] * qwh, axis=2)
        qd = jax.nn.softmax(logits, axis=1)
        qd = qd * msk
        qd = qd / (jnp.sum(qd, axis=1, keepdims=True) + 1e-6)
        wa_out[...] = qd
        ctx = jnp.sum(qd[:, :, None] * qwh, axis=1)
        rl = (jnp.dot(ctx, wr_ref[...], preferred_element_type=jnp.float32)
              + br_ref[...])
        rel_out[...] = jax.nn.sigmoid(rl)
    hop_full = (jnp.dot(qe, wh_ref[...], preferred_element_type=jnp.float32)
                + bh_ref[...])
    hop_dist = jax.nn.softmax(hop_full[:, :STEPS], axis=1)
    hop_ref[...] = hop_dist
    lane0 = jnp.zeros((B, LANES), jnp.float32)
    hb0_ref[...] = hop_dist[:, 0:1] + lane0
    hb1_ref[...] = hop_dist[:, 1:2] + lane0


_question_call = pl.pallas_call(
    _question_body,
    out_shape=(
        jax.ShapeDtypeStruct((B, L), jnp.float32),
        jax.ShapeDtypeStruct((B, L), jnp.float32),
        jax.ShapeDtypeStruct((B, R), jnp.float32),
        jax.ShapeDtypeStruct((B, R), jnp.float32),
        jax.ShapeDtypeStruct((B, STEPS), jnp.float32),
        jax.ShapeDtypeStruct((T // 128, 128), jnp.int32),
        jax.ShapeDtypeStruct((T // CH, CH // 2), jnp.int32),
        jax.ShapeDtypeStruct((B, LANES), jnp.float32),
        jax.ShapeDtypeStruct((B, LANES), jnp.float32),
    ),
)


def _sc_body(heads_hbm, rel0_hbm, rel1_hbm, sr_hbm, ob2_hbm, hb0_hbm, hb1_hbm,
             ent0_hbm, ent1_hbm, score_hbm,
             e_v, acc_v, r_v, sr_v, ob2_v, h_v, sem_a, sem_b, sem_w):
    b = lax.axis_index("s") * NC + lax.axis_index("c")

    zero16 = jnp.zeros((LANES,), jnp.float32)
    one16 = jnp.ones((LANES,), jnp.float32)

    def fire(c, slot, sem):
        pltpu.async_copy(sr_hbm.at[pl.ds(c * CH, CH)],
                         sr_v.at[pl.ds(slot * CH, CH)], sem)
        pltpu.async_copy(ob2_hbm.at[pl.ds(c * (CH // 2), CH // 2)],
                         ob2_v.at[pl.ds(slot * (CH // 2), CH // 2)], sem)

    def drain(sem):
        pltpu.make_async_copy(sr_hbm.at[pl.ds(0, CH)],
                              sr_v.at[pl.ds(0, CH)], sem).wait()
        pltpu.make_async_copy(ob2_hbm.at[pl.ds(0, CH // 2)],
                              ob2_v.at[pl.ds(0, CH // 2)], sem).wait()

    def compute(slot):
        soff = slot * CH
        ooff = slot * (CH // 2)

        @plsc.parallel_loop(0, CH // (2 * LANES), step=1, unroll=UNROLL)
        def _(k):
            o = k * LANES
            sr_a = sr_v[pl.ds(soff + o, LANES)]
            sr_b = sr_v[pl.ds(soff + CH // 2 + o, LANES)]
            ov = ob2_v[pl.ds(ooff + o, LANES)]
            ev_a = plsc.load_gather(e_v, [sr_a & 0xFFFF])
            rv_a = plsc.load_gather(r_v, [lax.shift_right_logical(sr_a, 16)])
            plsc.addupdate_scatter(acc_v, [ov & 0xFFFF], ev_a * rv_a)
            ev_b = plsc.load_gather(e_v, [sr_b & 0xFFFF])
            rv_b = plsc.load_gather(r_v, [lax.shift_right_logical(sr_b, 16)])
            plsc.addupdate_scatter(
                acc_v, [lax.shift_right_logical(ov, 16)], ev_b * rv_b)

    fire(0, 0, sem_a)
    pltpu.sync_copy(heads_hbm.at[pl.ds(b * E, E)], e_v)
    pltpu.sync_copy(hb0_hbm.at[pl.ds(b * LANES, LANES)],
                    h_v.at[pl.ds(0, LANES)])
    pltpu.sync_copy(hb1_hbm.at[pl.ds(b * LANES, LANES)],
                    h_v.at[pl.ds(LANES, LANES)])

    @plsc.parallel_loop(0, E // LANES, step=1, unroll=5)
    def _(j):
        acc_v[pl.ds(j * LANES, LANES)] = zero16

    def traverse(r_hbm):
        pltpu.sync_copy(r_hbm.at[pl.ds(b * R, R)], r_v)

        def pair_body(p, carry):
            c0 = 2 * p
            fire(c0 + 1, 1, sem_b)
            drain(sem_a)
            compute(0)
            fire(jnp.minimum(c0 + 2, NCH - 2), 0, sem_a)
            drain(sem_b)
            compute(1)
            return carry

        lax.fori_loop(0, NCH // 2, pair_body, 0)
        drain(sem_a)

    traverse(rel0_hbm)
    fire(0, 0, sem_a)

    @plsc.parallel_loop(0, E // LANES, step=1, unroll=5)
    def _(j):
        o = j * LANES
        x = acc_v[pl.ds(o, LANES)]
        e_v[pl.ds(o, LANES)] = jnp.minimum(x, one16)
        acc_v[pl.ds(o, LANES)] = zero16

    ent0_copy = pltpu.async_copy(e_v, ent0_hbm.at[pl.ds(b * E, E)], sem_w)

    traverse(rel1_hbm)
    ent0_copy.wait()
    h0v = h_v[pl.ds(0, LANES)]
    h1v = h_v[pl.ds(LANES, LANES)]

    @plsc.parallel_loop(0, E // LANES, step=1, unroll=5)
    def _(j):
        o = j * LANES
        e1 = jnp.minimum(acc_v[pl.ds(o, LANES)], one16)
        e0 = e_v[pl.ds(o, LANES)]
        acc_v[pl.ds(o, LANES)] = e1
        e_v[pl.ds(o, LANES)] = h0v * e0 + h1v * e1

    ent1_copy = pltpu.async_copy(acc_v, ent1_hbm.at[pl.ds(b * E, E)], sem_w)
    score_copy = pltpu.async_copy(e_v, score_hbm.at[pl.ds(b * E, E)], sem_w)
    ent1_copy.wait()
    score_copy.wait()


import functools


@functools.lru_cache(maxsize=1)
def _get_sc_follow():
    return pl.kernel(
        _sc_body,
        out_type=(
            jax.ShapeDtypeStruct((B * E,), jnp.float32),
            jax.ShapeDtypeStruct((B * E,), jnp.float32),
            jax.ShapeDtypeStruct((B * E,), jnp.float32),
        ),
        mesh=plsc.VectorSubcoreMesh(
            core_axis_name="c", subcore_axis_name="s",
            num_cores=NC, num_subcores=NS),
        compiler_params=pltpu.CompilerParams(needs_layout_passes=False),
        scratch_types=[
            pltpu.VMEM((E,), jnp.float32),
            pltpu.VMEM((E,), jnp.float32),
            pltpu.VMEM((R,), jnp.float32),
            pltpu.VMEM((2 * CH,), jnp.int32),
            pltpu.VMEM((CH,), jnp.int32),
            pltpu.VMEM((2 * LANES,), jnp.float32),
            pltpu.SemaphoreType.DMA,
            pltpu.SemaphoreType.DMA,
            pltpu.SemaphoreType.DMA,
        ],
    )


@jax.jit
def kernel(heads, q_embeddings, q_word_h, attention_mask,
           subj_idx, rel_idx, obj_idx,
           W_step0, b_step0, W_step1, b_step1,
           W_rel, b_rel, W_hop, b_hop):
    wa0, wa1, rel0, rel1, hop, packed_sr, packed_ob, hb0, hb1 = _question_call(
        q_embeddings, q_word_h, attention_mask,
        W_step0, b_step0.reshape(1, D), W_step1, b_step1.reshape(1, D),
        W_rel, b_rel.reshape(1, R), W_hop, b_hop.reshape(1, STEPS),
        subj_idx.reshape(T // 128, 128), rel_idx.reshape(T // 128, 128),
        obj_idx.reshape(T // CH, 2, CH // 2))

    ent0f, ent1f, scoref = _get_sc_follow()(
        heads.reshape(B * E), rel0.reshape(B * R), rel1.reshape(B * R),
        packed_sr.reshape(T), packed_ob.reshape(T // 2),
        hb0.reshape(B * LANES), hb1.reshape(B * LANES))
    ent0 = ent0f.reshape(B, E)
    ent1 = ent1f.reshape(B, E)
    e_score = scoref.reshape(B, E)
    return (e_score, wa0, wa1, rel0, rel1, ent0, ent1, hop)


# speedup vs baseline: 1.0207x; 1.0207x over previous
import jax
import jax.numpy as jnp
from jax import lax
from jax.experimental import pallas as pl
from jax.experimental.pallas import tpu as pltpu
from jax.experimental.pallas import tpu_sc as plsc

E = 50000
T = 800000
R = 512
D = 768
B = 32
L = 32
STEPS = 2

NC = 2
NS = 16
LANES = 16

CH = 8000
NCH = T // CH
UNROLL = 10


def _question_body(qe_ref, qwh_ref, mask_ref, w0_ref, b0_ref, w1_ref, b1_ref,
                   wr_ref, br_ref, wh_ref, bh_ref, sj_ref, rl_ref, ob_ref,
                   wa0_ref, wa1_ref, rel0_ref, rel1_ref, hop_ref, pack_ref,
                   ob2_ref):
    pack_ref[...] = sj_ref[...] | (rl_ref[...] << 16)
    ob2_ref[...] = ob_ref[:, 0, :] | (ob_ref[:, 1, :] << 16)
    qe = qe_ref[...]
    qwh = qwh_ref[...]
    msk = mask_ref[...]
    steps = ((w0_ref, b0_ref, wa0_ref, rel0_ref),
             (w1_ref, b1_ref, wa1_ref, rel1_ref))
    for w_ref, b_ref, wa_out, rel_out in steps:
        cq = jnp.tanh(
            jnp.dot(qe, w_ref[...], preferred_element_type=jnp.float32)
            + b_ref[...])
        logits = jnp.sum(cq[:, None, :] * qwh, axis=2)
        qd = jax.nn.softmax(logits, axis=1)
        qd = qd * msk
        qd = qd / (jnp.sum(qd, axis=1, keepdims=True) + 1e-6)
        wa_out[...] = qd
        ctx = jnp.sum(qd[:, :, None] * qwh, axis=1)
        rl = (jnp.dot(ctx, wr_ref[...], preferred_element_type=jnp.float32)
              + br_ref[...])
        rel_out[...] = jax.nn.sigmoid(rl)
    hop_full = (jnp.dot(qe, wh_ref[...], preferred_element_type=jnp.float32)
                + bh_ref[...])
    hop_ref[...] = jax.nn.softmax(hop_full[:, :STEPS], axis=1)


_question_call = pl.pallas_call(
    _question_body,
    out_shape=(
        jax.ShapeDtypeStruct((B, L), jnp.float32),
        jax.ShapeDtypeStruct((B, L), jnp.float32),
        jax.ShapeDtypeStruct((B, R), jnp.float32),
        jax.ShapeDtypeStruct((B, R), jnp.float32),
        jax.ShapeDtypeStruct((B, STEPS), jnp.float32),
        jax.ShapeDtypeStruct((T // 128, 128), jnp.int32),
        jax.ShapeDtypeStruct((T // CH, CH // 2), jnp.int32),
    ),
)


def _sc_body(heads_hbm, rel0_hbm, rel1_hbm, sr_hbm, ob2_hbm,
             ent0_hbm, ent1_hbm,
             e_v, acc_v, r_v, sr_v, ob2_v, sem_a, sem_b, sem_w):
    b = lax.axis_index("s") * NC + lax.axis_index("c")

    zero16 = jnp.zeros((LANES,), jnp.float32)
    one16 = jnp.ones((LANES,), jnp.float32)

    def fire(c, slot, sem):
        pltpu.async_copy(sr_hbm.at[pl.ds(c * CH, CH)],
                         sr_v.at[pl.ds(slot * CH, CH)], sem)
        pltpu.async_copy(ob2_hbm.at[pl.ds(c * (CH // 2), CH // 2)],
                         ob2_v.at[pl.ds(slot * (CH // 2), CH // 2)], sem)

    def drain(sem):
        pltpu.make_async_copy(sr_hbm.at[pl.ds(0, CH)],
                              sr_v.at[pl.ds(0, CH)], sem).wait()
        pltpu.make_async_copy(ob2_hbm.at[pl.ds(0, CH // 2)],
                              ob2_v.at[pl.ds(0, CH // 2)], sem).wait()

    def compute(slot):
        soff = slot * CH
        ooff = slot * (CH // 2)

        @plsc.parallel_loop(0, CH // (2 * LANES), step=1, unroll=UNROLL)
        def _(k):
            o = k * LANES
            sr_a = sr_v[pl.ds(soff + o, LANES)]
            sr_b = sr_v[pl.ds(soff + CH // 2 + o, LANES)]
            ov = ob2_v[pl.ds(ooff + o, LANES)]
            ev_a = plsc.load_gather(e_v, [sr_a & 0xFFFF])
            rv_a = plsc.load_gather(r_v, [lax.shift_right_logical(sr_a, 16)])
            plsc.addupdate_scatter(acc_v, [ov & 0xFFFF], ev_a * rv_a)
            ev_b = plsc.load_gather(e_v, [sr_b & 0xFFFF])
            rv_b = plsc.load_gather(r_v, [lax.shift_right_logical(sr_b, 16)])
            plsc.addupdate_scatter(
                acc_v, [lax.shift_right_logical(ov, 16)], ev_b * rv_b)

    fire(0, 0, sem_a)
    pltpu.sync_copy(heads_hbm.at[pl.ds(b * E, E)], e_v)

    @plsc.parallel_loop(0, E // LANES, step=1, unroll=5)
    def _(j):
        acc_v[pl.ds(j * LANES, LANES)] = zero16

    def traverse(r_hbm):
        pltpu.sync_copy(r_hbm.at[pl.ds(b * R, R)], r_v)

        def pair_body(p, carry):
            c0 = 2 * p
            fire(c0 + 1, 1, sem_b)
            drain(sem_a)
            compute(0)
            fire(jnp.minimum(c0 + 2, NCH - 2), 0, sem_a)
            drain(sem_b)
            compute(1)
            return carry

        lax.fori_loop(0, NCH // 2, pair_body, 0)
        drain(sem_a)

    traverse(rel0_hbm)
    fire(0, 0, sem_a)

    @plsc.parallel_loop(0, E // LANES, step=1, unroll=5)
    def _(j):
        o = j * LANES
        x = acc_v[pl.ds(o, LANES)]
        e_v[pl.ds(o, LANES)] = jnp.minimum(x, one16)
        acc_v[pl.ds(o, LANES)] = zero16

    ent0_copy = pltpu.async_copy(e_v, ent0_hbm.at[pl.ds(b * E, E)], sem_w)

    traverse(rel1_hbm)
    ent0_copy.wait()

    @plsc.parallel_loop(0, E // LANES, step=1, unroll=5)
    def _(j):
        o = j * LANES
        e_v[pl.ds(o, LANES)] = jnp.minimum(acc_v[pl.ds(o, LANES)], one16)

    pltpu.sync_copy(e_v, ent1_hbm.at[pl.ds(b * E, E)])


import functools


@functools.lru_cache(maxsize=1)
def _get_sc_follow():
    return pl.kernel(
        _sc_body,
        out_type=(
            jax.ShapeDtypeStruct((B * E,), jnp.float32),
            jax.ShapeDtypeStruct((B * E,), jnp.float32),
        ),
        mesh=plsc.VectorSubcoreMesh(
            core_axis_name="c", subcore_axis_name="s",
            num_cores=NC, num_subcores=NS),
        compiler_params=pltpu.CompilerParams(needs_layout_passes=False),
        scratch_types=[
            pltpu.VMEM((E,), jnp.float32),
            pltpu.VMEM((E,), jnp.float32),
            pltpu.VMEM((R,), jnp.float32),
            pltpu.VMEM((2 * CH,), jnp.int32),
            pltpu.VMEM((CH,), jnp.int32),
            pltpu.SemaphoreType.DMA,
            pltpu.SemaphoreType.DMA,
            pltpu.SemaphoreType.DMA,
        ],
    )


def _combine_body(ent0_ref, ent1_ref, hop_ref, out_ref):
    h0 = hop_ref[:, 0:1]
    h1 = hop_ref[:, 1:2]
    out_ref[...] = h0 * ent0_ref[...] + h1 * ent1_ref[...]


_combine_call = pl.pallas_call(
    _combine_body,
    grid=(4,),
    in_specs=[
        pl.BlockSpec((B // 4, E), lambda i: (i, 0)),
        pl.BlockSpec((B // 4, E), lambda i: (i, 0)),
        pl.BlockSpec((B // 4, STEPS), lambda i: (i, 0)),
    ],
    out_specs=pl.BlockSpec((B // 4, E), lambda i: (i, 0)),
    out_shape=jax.ShapeDtypeStruct((B, E), jnp.float32),
)


@jax.jit
def kernel(heads, q_embeddings, q_word_h, attention_mask,
           subj_idx, rel_idx, obj_idx,
           W_step0, b_step0, W_step1, b_step1,
           W_rel, b_rel, W_hop, b_hop):
    wa0, wa1, rel0, rel1, hop, packed_sr, packed_ob = _question_call(
        q_embeddings, q_word_h, attention_mask,
        W_step0, b_step0.reshape(1, D), W_step1, b_step1.reshape(1, D),
        W_rel, b_rel.reshape(1, R), W_hop, b_hop.reshape(1, STEPS),
        subj_idx.reshape(T // 128, 128), rel_idx.reshape(T // 128, 128),
        obj_idx.reshape(T // CH, 2, CH // 2))

    ent0f, ent1f = _get_sc_follow()(
        heads.reshape(B * E), rel0.reshape(B * R), rel1.reshape(B * R),
        packed_sr.reshape(T), packed_ob.reshape(T // 2))
    ent0 = ent0f.reshape(B, E)
    ent1 = ent1f.reshape(B, E)

    e_score = _combine_call(ent0, ent1, hop)
    return (e_score, wa0, wa1, rel0, rel1, ent0, ent1, hop)
